# Initial kernel scaffold; baseline (speedup 1.0000x reference)
import jax
import jax.numpy as jnp
from jax import lax
from jax.experimental import pallas as pl

D_POINTS = 64
D_MODEL = 128
K = 24
POS_DIM = 60


def _sincos_3d(embed_dim, pts):
    d = embed_dim // 3
    omega = jnp.arange(d // 2, dtype=jnp.float32) / (d / 2.0)
    omega = 1.0 / (10000.0 ** omega)
    parts = []
    for i in range(3):
        out = pts[..., i][..., None] * omega
        parts.append(jnp.concatenate([jnp.sin(out), jnp.cos(out)], axis=-1))
    return jnp.concatenate(parts, axis=-1)


def kernel(features, xyz, W1, b1, W2, b2, Wd1, bd1, Wd2, bd2, Wg1, bg1, Wg2, bg2):
    P = lax.Precision.HIGHEST
    q2 = jnp.sum(xyz ** 2, -1)
    qk = jnp.matmul(xyz, jnp.swapaxes(xyz, -1, -2), precision=P)
    dists = q2[..., :, None] + q2[..., None, :] - 2.0 * qk
    _, knn_idx = lax.top_k(-dists, K)
    knn_xyz = jax.vmap(lambda p, i: p[i])(xyz, knn_idx)
    x = jnp.matmul(features, W1, precision=P) + b1
    q = x
    kf = jax.vmap(lambda p, i: p[i])(x, knn_idx)
    v = kf
    g_xyz = xyz[:, :, None, :] - knn_xyz
    g_embed = _sincos_3d(POS_DIM, g_xyz)
    pos_enc = jnp.matmul(jax.nn.relu(jnp.matmul(g_embed, Wd1, precision=P) + bd1), Wd2, precision=P) + bd2
    pre = q[:, :, None, :] - kf + pos_enc
    attn = jnp.matmul(jax.nn.relu(jnp.matmul(pre, Wg1, precision=P) + bg1), Wg2, precision=P) + bg2
    attn = jax.nn.softmax(attn / jnp.sqrt(jnp.float32(kf.shape[-1])), axis=-2)
    res = jnp.einsum('bmnf,bmnf->bmf', attn, v + pos_enc)
    res = jnp.matmul(res, W2, precision=P) + b2 + x
    return (res, attn)


# speedup vs baseline: 1.1101x; 1.1101x over previous
import jax
import jax.numpy as jnp
from jax import lax
from jax.experimental import pallas as pl

D_POINTS = 64
D_MODEL = 128
K = 24
POS_DIM = 60


def _sincos_3d(embed_dim, pts):
    d = embed_dim // 3
    omega = jnp.arange(d // 2, dtype=jnp.float32) / (d / 2.0)
    omega = 1.0 / (10000.0 ** omega)
    parts = []
    for i in range(3):
        out = pts[..., i][..., None] * omega
        parts.append(jnp.concatenate([jnp.sin(out), jnp.cos(out)], axis=-1))
    return jnp.concatenate(parts, axis=-1)


def kernel(features, xyz, W1, b1, W2, b2, Wd1, bd1, Wd2, bd2, Wg1, bg1, Wg2, bg2):
    pass
    q2 = jnp.sum(xyz ** 2, -1)
    qk = jnp.matmul(xyz.astype(jnp.bfloat16), jnp.swapaxes(xyz, -1, -2).astype(jnp.bfloat16),
                    preferred_element_type=jnp.float32)
    dists = q2[..., :, None] + q2[..., None, :] - 2.0 * qk
    _, knn_idx = lax.top_k(-dists, K)
    knn_xyz = jax.vmap(lambda p, i: p[i])(xyz, knn_idx)
    x = jnp.matmul(features, W1) + b1
    q = x
    kf = jax.vmap(lambda p, i: p[i])(x, knn_idx)
    v = kf
    g_xyz = xyz[:, :, None, :] - knn_xyz
    g_embed = _sincos_3d(POS_DIM, g_xyz)
    pos_enc = jnp.matmul(jax.nn.relu(jnp.matmul(g_embed, Wd1) + bd1), Wd2) + bd2
    pre = q[:, :, None, :] - kf + pos_enc
    attn = jnp.matmul(jax.nn.relu(jnp.matmul(pre, Wg1) + bg1), Wg2) + bg2
    attn = jax.nn.softmax(attn / jnp.sqrt(jnp.float32(kf.shape[-1])), axis=-2)
    res = jnp.einsum('bmnf,bmnf->bmf', attn, v + pos_enc)
    res = jnp.matmul(res, W2) + b2 + x
    return (res, attn)


# speedup vs baseline: 12.5651x; 11.3192x over previous
import functools

import jax
import jax.numpy as jnp
from jax import lax
from jax.experimental import pallas as pl
from jax.experimental.pallas import tpu as pltpu
from jax.experimental.pallas import tpu_sc as plsc

D_POINTS = 64
D_MODEL = 128
K = 24
POS_DIM = 60

B = 4
N = 4096
MA = 128
MC = 128
NSLAB = N // 128
IMAX = 0x7FFFFFFF


def _bdot(a, b, dims=(((1,), (0,)), ((), ()))):
    return lax.dot_general(a.astype(jnp.bfloat16), b.astype(jnp.bfloat16),
                           dims, preferred_element_type=jnp.float32)


def _topk_body(xyzq_ref, xyzk_ref, feat_ref, w1_ref, b1_ref, x_ref, idx_ref):
    b = pl.program_id(0)
    q = xyzq_ref[0]
    kx = xyzk_ref[0]
    q2 = jnp.sum(q * q, axis=-1)
    k2 = jnp.sum(kx * kx, axis=-1)
    qk = _bdot(q, kx, (((1,), (1,)), ((), ())))
    d = q2[:, None] + k2[None, :] - 2.0 * qk

    bits = lax.bitcast_convert_type(d, jnp.int32)
    ikey = bits ^ (lax.shift_right_arithmetic(bits, 31) & jnp.int32(0x7FFFFFFF))

    m1 = m2 = m3 = m4 = jnp.full((MA, 128), IMAX, jnp.int32)
    for j in range(NSLAB):
        s = (ikey[:, j * 128:(j + 1) * 128] & jnp.int32(-32)) | jnp.int32(j)
        lo = jnp.minimum(m1, s)
        hi = jnp.maximum(m1, s)
        m1 = lo
        lo = jnp.minimum(m2, hi)
        hi = jnp.maximum(m2, hi)
        m2 = lo
        lo = jnp.minimum(m3, hi)
        hi = jnp.maximum(m3, hi)
        m3 = lo
        m4 = jnp.minimum(m4, hi)

    lane = lax.broadcasted_iota(jnp.int32, (MA, 128), 1)
    kcol = lax.broadcasted_iota(jnp.int32, (MA, K), 1)
    out_idx = jnp.zeros((MA, K), jnp.int32)
    for kk in range(K):
        w = jnp.min(m1, axis=1, keepdims=True)
        eq = m1 == w
        l = jnp.min(jnp.where(eq, lane, jnp.int32(128)), axis=1, keepdims=True)
        gidx = (w & jnp.int32(31)) * 128 + l
        out_idx = jnp.where(kcol == kk, gidx, out_idx)
        eql = lane == l
        m1 = jnp.where(eql, m2, m1)
        m2 = jnp.where(eql, m3, m2)
        m3 = jnp.where(eql, m4, m3)
        m4 = jnp.where(eql, IMAX, m4)

    idx_ref[0] = out_idx + b * N
    x_ref[0] = _bdot(feat_ref[0], w1_ref[...]) + b1_ref[...]


def _run_topk(xyz16, features, W1, b1):
    grid = (B, N // MA)
    return pl.pallas_call(
        _topk_body,
        grid=grid,
        in_specs=[
            pl.BlockSpec((1, MA, 16), lambda b, i: (b, i, 0)),
            pl.BlockSpec((1, N, 16), lambda b, i: (b, 0, 0)),
            pl.BlockSpec((1, MA, D_POINTS), lambda b, i: (b, i, 0)),
            pl.BlockSpec((D_POINTS, D_MODEL), lambda b, i: (0, 0)),
            pl.BlockSpec((1, D_MODEL), lambda b, i: (0, 0)),
        ],
        out_specs=[
            pl.BlockSpec((1, MA, D_MODEL), lambda b, i: (b, i, 0)),
            pl.BlockSpec((1, MA, K), lambda b, i: (b, i, 0)),
        ],
        out_shape=[
            jax.ShapeDtypeStruct((B, N, D_MODEL), jnp.float32),
            jax.ShapeDtypeStruct((B, N, K), jnp.int32),
        ],
        compiler_params=pltpu.CompilerParams(
            dimension_semantics=("parallel", "parallel")),
    )(xyz16, xyz16, features, W1, b1)


NC_SC = 2
NS_SC = 16
NW = NC_SC * NS_SC
ROWS = B * N * K
RPW = ROWS // NW
CH = 128
NCHUNK = RPW // CH


def _gather_sc(x2, xyz16t, gidx):
    mesh = plsc.VectorSubcoreMesh(core_axis_name="c", subcore_axis_name="s")

    @functools.partial(
        pl.kernel, mesh=mesh,
        out_type=[
            jax.ShapeDtypeStruct((ROWS, D_MODEL), jnp.float32),
            jax.ShapeDtypeStruct((ROWS, 128), jnp.float32),
        ],
        scratch_types=[
            pltpu.VMEM((CH,), jnp.int32),
            pltpu.VMEM((CH, D_MODEL), jnp.float32),
            pltpu.VMEM((CH, 128), jnp.float32),
            pltpu.SemaphoreType.DMA,
            pltpu.SemaphoreType.DMA,
        ],
    )
    def gather_kernel(x2_hbm, xyzt_hbm, gidx_hbm, kf_hbm, kx_hbm,
                      idxv, kfv, kxv, sem0, sem1):
        wid = lax.axis_index("s") * NC_SC + lax.axis_index("c")
        base0 = wid * RPW

        def body(i, carry):
            base = base0 + i * CH
            pltpu.sync_copy(gidx_hbm.at[pl.ds(base, CH)], idxv)
            cp0 = pltpu.async_copy(x2_hbm.at[idxv], kfv, sem0)
            cp1 = pltpu.async_copy(xyzt_hbm.at[idxv], kxv, sem1)
            cp0.wait()
            cp1.wait()
            pltpu.sync_copy(kfv, kf_hbm.at[pl.ds(base, CH)])
            pltpu.sync_copy(kxv, kx_hbm.at[pl.ds(base, CH)])
            return carry

        lax.fori_loop(0, NCHUNK, body, 0)

    return gather_kernel(x2, xyz16t, gidx)


def _mlp_body(x_ref, xyzq_ref, kf_ref, kx_ref, posw_ref,
              wd1_ref, bd1_ref, wd2_ref, bd2_ref,
              wg1_ref, bg1_ref, wg2_ref, bg2_ref, w2_ref, b2_ref,
              attn_ref, res_ref):
    MK = MC * K
    xq = x_ref[0]
    qxyz = xyzq_ref[0]
    kf = kf_ref[...]
    kxyz = kx_ref[...][:, :16]

    qxyz_rep = jnp.broadcast_to(qxyz[:, None, :], (MC, K, 16)).reshape(MK, 16)
    g = qxyz_rep - kxyz

    posw = posw_ref[...]
    ang = (g[:, 0:1] * posw[0:1, :]
           + g[:, 1:2] * posw[1:2, :]
           + g[:, 2:3] * posw[2:3, :])
    lane64 = lax.broadcasted_iota(jnp.int32, (MK, 64), 1)
    emb = jnp.where(lane64 < 32, jnp.sin(ang), jnp.cos(ang))

    pos_enc = _bdot(jnp.maximum(_bdot(emb, wd1_ref[...]) + bd1_ref[...], 0.0),
                    wd2_ref[...]) + bd2_ref[...]

    xq_rep = jnp.broadcast_to(xq[:, None, :], (MC, K, D_MODEL)).reshape(MK, D_MODEL)
    pre = xq_rep - kf + pos_enc
    a1 = _bdot(jnp.maximum(_bdot(pre, wg1_ref[...]) + bg1_ref[...], 0.0),
               wg2_ref[...]) + bg2_ref[...]
    a1 = a1 / jnp.sqrt(jnp.float32(D_MODEL))

    a3 = a1.reshape(MC, K, D_MODEL)
    mx = jnp.max(a3, axis=1, keepdims=True)
    e = jnp.exp(a3 - mx)
    s = jnp.sum(e, axis=1, keepdims=True)
    attn = e / s
    attn_ref[0] = attn

    prod = attn.reshape(MK, D_MODEL) * (kf + pos_enc)
    red = jnp.sum(prod.reshape(MC, K, D_MODEL), axis=1)
    res_ref[0] = _bdot(red, w2_ref[...]) + b2_ref[...] + xq


def _run_mlp(x, xyz16, kf, kxyz, posw, Wd1p, bd1, Wd2, bd2, Wg1, bg1, Wg2, bg2, W2, b2):
    grid = (B, N // MC)
    nblk = N // MC
    return pl.pallas_call(
        _mlp_body,
        grid=grid,
        in_specs=[
            pl.BlockSpec((1, MC, D_MODEL), lambda b, i: (b, i, 0)),
            pl.BlockSpec((1, MC, 16), lambda b, i: (b, i, 0)),
            pl.BlockSpec((MC * K, D_MODEL), lambda b, i: (b * nblk + i, 0)),
            pl.BlockSpec((MC * K, 128), lambda b, i: (b * nblk + i, 0)),
            pl.BlockSpec((3, 64), lambda b, i: (0, 0)),
            pl.BlockSpec((64, D_MODEL), lambda b, i: (0, 0)),
            pl.BlockSpec((1, D_MODEL), lambda b, i: (0, 0)),
            pl.BlockSpec((D_MODEL, D_MODEL), lambda b, i: (0, 0)),
            pl.BlockSpec((1, D_MODEL), lambda b, i: (0, 0)),
            pl.BlockSpec((D_MODEL, D_MODEL), lambda b, i: (0, 0)),
            pl.BlockSpec((1, D_MODEL), lambda b, i: (0, 0)),
            pl.BlockSpec((D_MODEL, D_MODEL), lambda b, i: (0, 0)),
            pl.BlockSpec((1, D_MODEL), lambda b, i: (0, 0)),
            pl.BlockSpec((D_MODEL, D_MODEL), lambda b, i: (0, 0)),
            pl.BlockSpec((1, D_MODEL), lambda b, i: (0, 0)),
        ],
        out_specs=[
            pl.BlockSpec((1, MC, K, D_MODEL), lambda b, i: (b, i, 0, 0)),
            pl.BlockSpec((1, MC, D_MODEL), lambda b, i: (b, i, 0)),
        ],
        out_shape=[
            jax.ShapeDtypeStruct((B, N, K, D_MODEL), jnp.float32),
            jax.ShapeDtypeStruct((B, N, D_MODEL), jnp.float32),
        ],
        compiler_params=pltpu.CompilerParams(
            dimension_semantics=("parallel", "parallel")),
    )(x, xyz16, kf, kxyz, posw, Wd1p, bd1, Wd2, bd2, Wg1, bg1, Wg2, bg2, W2, b2)


def kernel(features, xyz, W1, b1, W2, b2, Wd1, bd1, Wd2, bd2, Wg1, bg1, Wg2, bg2):
    f32 = jnp.float32
    xyz16 = jnp.concatenate(
        [xyz, jnp.zeros((B, N, 13), f32)], axis=-1)

    x, gidx = _run_topk(xyz16, features, W1, b1.reshape(1, D_MODEL))

    xyzt = jnp.concatenate(
        [xyz16.reshape(B * N, 16), jnp.zeros((B * N, 112), f32)], axis=-1)
    kf, kxyz = _gather_sc(x.reshape(B * N, D_MODEL), xyzt, gidx.reshape(ROWS))

    d2 = POS_DIM // 3
    omega = jnp.arange(d2 // 2, dtype=f32) / (d2 / 2.0)
    omega = 1.0 / (10000.0 ** omega)
    col = jnp.arange(64)
    axis_of_col = (col % 32) // 10
    freq_of_col = (col % 32) % 10
    valid = (col % 32) < 30
    posw = jnp.where(valid[None, :] & (axis_of_col[None, :] == jnp.arange(3)[:, None]),
                     omega[freq_of_col][None, :], 0.0).astype(f32)

    src = jnp.where(valid, 20 * axis_of_col + jnp.where(col < 32, 0, 10) + freq_of_col, 0)
    Wd1p = jnp.where(valid[:, None], Wd1[src], 0.0)

    res, attn_unused_shape = None, None
    attn, res = _run_mlp(x, xyz16, kf, kxyz, posw, Wd1p,
                         bd1.reshape(1, D_MODEL), Wd2, bd2.reshape(1, D_MODEL),
                         Wg1, bg1.reshape(1, D_MODEL), Wg2, bg2.reshape(1, D_MODEL),
                         W2, b2.reshape(1, D_MODEL))
    return (res, attn)


# speedup vs baseline: 24.3465x; 1.9376x over previous
import functools

import jax
import jax.numpy as jnp
from jax import lax
from jax.experimental import pallas as pl
from jax.experimental.pallas import tpu as pltpu
from jax.experimental.pallas import tpu_sc as plsc

D_POINTS = 64
D_MODEL = 128
K = 24
POS_DIM = 60

B = 4
N = 4096
MA = 512
MC = 256
NSLAB = N // 128
IMAX = 0x7FFFFFFF


def _bdot(a, b, dims=(((1,), (0,)), ((), ()))):
    return lax.dot_general(a.astype(jnp.bfloat16), b.astype(jnp.bfloat16),
                           dims, preferred_element_type=jnp.float32)


def _topk_body(xyzq_ref, xyzk_ref, feat_ref, w1_ref, b1_ref, posw_ref,
               x_ref, idx_ref, tab_ref):
    b = pl.program_id(0)
    q = xyzq_ref[0]
    kx = xyzk_ref[0]
    q2 = jnp.sum(q * q, axis=-1)
    k2 = jnp.sum(kx * kx, axis=-1)
    qk = _bdot(q, kx, (((1,), (1,)), ((), ())))
    d = q2[:, None] + k2[None, :] - 2.0 * qk

    bits = lax.bitcast_convert_type(d, jnp.int32)
    ikey = bits ^ (lax.shift_right_arithmetic(bits, 31) & jnp.int32(0x7FFFFFFF))

    m1 = m2 = m3 = m4 = jnp.full((MA, 128), IMAX, jnp.int32)
    for j in range(NSLAB):
        s = (ikey[:, j * 128:(j + 1) * 128] & jnp.int32(-32)) | jnp.int32(j)
        lo = jnp.minimum(m1, s)
        hi = jnp.maximum(m1, s)
        m1 = lo
        lo = jnp.minimum(m2, hi)
        hi = jnp.maximum(m2, hi)
        m2 = lo
        lo = jnp.minimum(m3, hi)
        hi = jnp.maximum(m3, hi)
        m3 = lo
        m4 = jnp.minimum(m4, hi)

    lane = lax.broadcasted_iota(jnp.int32, (MA, 128), 1)
    kcol = lax.broadcasted_iota(jnp.int32, (MA, K), 1)
    out_idx = jnp.zeros((MA, K), jnp.int32)
    for kk in range(K):
        w = jnp.min(m1, axis=1, keepdims=True)
        eq = m1 == w
        l = jnp.min(jnp.where(eq, lane, jnp.int32(128)), axis=1, keepdims=True)
        gidx = (w & jnp.int32(31)) * 128 + l
        out_idx = jnp.where(kcol == kk, gidx, out_idx)
        eql = lane == l
        m1 = jnp.where(eql, m2, m1)
        m2 = jnp.where(eql, m3, m2)
        m3 = jnp.where(eql, m4, m3)
        m4 = jnp.where(eql, IMAX, m4)

    idx_ref[0] = out_idx + b * N

    posw = posw_ref[...]
    ang = (q[:, 0:1] * posw[0:1, :]
           + q[:, 1:2] * posw[1:2, :]
           + q[:, 2:3] * posw[2:3, :])
    sn = jnp.sin(ang)
    cs = jnp.cos(ang)
    tab = jnp.where(lane < 32, q, jnp.where(lane < 64, sn,
                    jnp.where(lane < 96, cs, 0.0)))
    tab_ref[0] = tab

    x_ref[0] = _bdot(feat_ref[0], w1_ref[...]) + b1_ref[...]


def _run_topk(xyz128, features, W1, b1, posw):
    grid = (B, N // MA)
    return pl.pallas_call(
        _topk_body,
        grid=grid,
        in_specs=[
            pl.BlockSpec((1, MA, 128), lambda b, i: (b, i, 0)),
            pl.BlockSpec((1, N, 128), lambda b, i: (b, 0, 0)),
            pl.BlockSpec((1, MA, D_POINTS), lambda b, i: (b, i, 0)),
            pl.BlockSpec((D_POINTS, D_MODEL), lambda b, i: (0, 0)),
            pl.BlockSpec((1, D_MODEL), lambda b, i: (0, 0)),
            pl.BlockSpec((8, 128), lambda b, i: (0, 0)),
        ],
        out_specs=[
            pl.BlockSpec((1, MA, D_MODEL), lambda b, i: (b, i, 0)),
            pl.BlockSpec((1, MA, K), lambda b, i: (b, i, 0)),
            pl.BlockSpec((1, MA, 128), lambda b, i: (b, i, 0)),
        ],
        out_shape=[
            jax.ShapeDtypeStruct((B, N, D_MODEL), jnp.float32),
            jax.ShapeDtypeStruct((B, N, K), jnp.int32),
            jax.ShapeDtypeStruct((B, N, 128), jnp.float32),
        ],
        compiler_params=pltpu.CompilerParams(
            dimension_semantics=("parallel", "parallel")),
    )(xyz128, xyz128, features, W1, b1, posw)


NC_SC = 2
NS_SC = 16
NW = NC_SC * NS_SC
ROWS = B * N * K
RPW = ROWS // NW
CH = 128
NCHUNK = RPW // CH


def _gather_sc(x2, tab2, gidx):
    mesh = plsc.VectorSubcoreMesh(core_axis_name="c", subcore_axis_name="s")

    @functools.partial(
        pl.kernel, mesh=mesh,
        out_type=[
            jax.ShapeDtypeStruct((ROWS, D_MODEL), jnp.float32),
            jax.ShapeDtypeStruct((ROWS, 128), jnp.float32),
        ],
        scratch_types=[
            pltpu.VMEM((CH,), jnp.int32),
            pltpu.VMEM((CH, D_MODEL), jnp.float32),
            pltpu.VMEM((CH, 128), jnp.float32),
            pltpu.SemaphoreType.DMA,
            pltpu.SemaphoreType.DMA,
        ],
    )
    def gather_kernel(x2_hbm, tab_hbm, gidx_hbm, kf_hbm, ktab_hbm,
                      idxv, kfv, ktv, sem0, sem1):
        wid = lax.axis_index("s") * NC_SC + lax.axis_index("c")
        base0 = wid * RPW

        def body(i, carry):
            base = base0 + i * CH
            pltpu.sync_copy(gidx_hbm.at[pl.ds(base, CH)], idxv)
            cp0 = pltpu.async_copy(x2_hbm.at[idxv], kfv, sem0)
            cp1 = pltpu.async_copy(tab_hbm.at[idxv], ktv, sem1)
            cp0.wait()
            cp1.wait()
            pltpu.sync_copy(kfv, kf_hbm.at[pl.ds(base, CH)])
            pltpu.sync_copy(ktv, ktab_hbm.at[pl.ds(base, CH)])
            return carry

        lax.fori_loop(0, NCHUNK, body, 0)

    return gather_kernel(x2, tab2, gidx)


def _mlp_body(x_ref, tabq_ref, kf_ref, kt_ref,
              wd1s_ref, wd1c_ref, bd1_ref, wd2_ref, bd2_ref,
              wg1_ref, bg1_ref, wg2_ref, bg2_ref, w2_ref, b2_ref,
              attn_ref, res_ref):
    MK = MC * K
    xq = x_ref[0]
    tq = tabq_ref[0]
    kf = kf_ref[...]
    kt = kt_ref[...]

    tq_rep = jnp.broadcast_to(tq[:, None, :], (MC, K, 128)).reshape(MK, 128)
    qsin = tq_rep[:, 32:64]
    qcos = tq_rep[:, 64:96]
    ksin = kt[:, 32:64]
    kcos = kt[:, 64:96]
    emb_sin = qsin * kcos - qcos * ksin
    emb_cos = qcos * kcos + qsin * ksin

    pos_pre = _bdot(emb_sin, wd1s_ref[...]) + _bdot(emb_cos, wd1c_ref[...]) + bd1_ref[...]
    pos_enc = _bdot(jnp.maximum(pos_pre, 0.0), wd2_ref[...]) + bd2_ref[...]

    xq_rep = jnp.broadcast_to(xq[:, None, :], (MC, K, D_MODEL)).reshape(MK, D_MODEL)
    pre = xq_rep - kf + pos_enc
    a1 = _bdot(jnp.maximum(_bdot(pre, wg1_ref[...]) + bg1_ref[...], 0.0),
               wg2_ref[...]) + bg2_ref[...]
    a1 = a1 / jnp.sqrt(jnp.float32(D_MODEL))

    a3 = a1.reshape(MC, K, D_MODEL)
    mx = jnp.max(a3, axis=1, keepdims=True)
    e = jnp.exp(a3 - mx)
    s = jnp.sum(e, axis=1, keepdims=True)
    attn = e / s
    attn_ref[0] = attn

    prod = attn.reshape(MK, D_MODEL) * (kf + pos_enc)
    red = jnp.sum(prod.reshape(MC, K, D_MODEL), axis=1)
    res_ref[0] = _bdot(red, w2_ref[...]) + b2_ref[...] + xq


def _run_mlp(x, tab, kf, ktab, Wd1s, Wd1c, bd1, Wd2, bd2, Wg1, bg1, Wg2, bg2, W2, b2):
    grid = (B, N // MC)
    nblk = N // MC
    return pl.pallas_call(
        _mlp_body,
        grid=grid,
        in_specs=[
            pl.BlockSpec((1, MC, D_MODEL), lambda b, i: (b, i, 0)),
            pl.BlockSpec((1, MC, 128), lambda b, i: (b, i, 0)),
            pl.BlockSpec((MC * K, D_MODEL), lambda b, i: (b * nblk + i, 0)),
            pl.BlockSpec((MC * K, 128), lambda b, i: (b * nblk + i, 0)),
            pl.BlockSpec((32, D_MODEL), lambda b, i: (0, 0)),
            pl.BlockSpec((32, D_MODEL), lambda b, i: (0, 0)),
            pl.BlockSpec((1, D_MODEL), lambda b, i: (0, 0)),
            pl.BlockSpec((D_MODEL, D_MODEL), lambda b, i: (0, 0)),
            pl.BlockSpec((1, D_MODEL), lambda b, i: (0, 0)),
            pl.BlockSpec((D_MODEL, D_MODEL), lambda b, i: (0, 0)),
            pl.BlockSpec((1, D_MODEL), lambda b, i: (0, 0)),
            pl.BlockSpec((D_MODEL, D_MODEL), lambda b, i: (0, 0)),
            pl.BlockSpec((1, D_MODEL), lambda b, i: (0, 0)),
            pl.BlockSpec((D_MODEL, D_MODEL), lambda b, i: (0, 0)),
            pl.BlockSpec((1, D_MODEL), lambda b, i: (0, 0)),
        ],
        out_specs=[
            pl.BlockSpec((1, MC, K, D_MODEL), lambda b, i: (b, i, 0, 0)),
            pl.BlockSpec((1, MC, D_MODEL), lambda b, i: (b, i, 0)),
        ],
        out_shape=[
            jax.ShapeDtypeStruct((B, N, K, D_MODEL), jnp.float32),
            jax.ShapeDtypeStruct((B, N, D_MODEL), jnp.float32),
        ],
        compiler_params=pltpu.CompilerParams(
            dimension_semantics=("parallel", "parallel")),
    )(x, tab, kf, ktab, Wd1s, Wd1c, bd1, Wd2, bd2, Wg1, bg1, Wg2, bg2, W2, b2)


def kernel(features, xyz, W1, b1, W2, b2, Wd1, bd1, Wd2, bd2, Wg1, bg1, Wg2, bg2):
    f32 = jnp.float32
    xyz128 = jnp.concatenate(
        [xyz, jnp.zeros((B, N, 125), f32)], axis=-1)

    d2 = POS_DIM // 3
    omega = jnp.arange(d2 // 2, dtype=f32) / (d2 / 2.0)
    omega = 1.0 / (10000.0 ** omega)
    col = jnp.arange(128)
    sub = col % 32
    a_of = jnp.clip(sub // 10, 0, 2)
    f_of = sub % 10
    in_band = ((col >= 32) & (col < 128 - 32)) & (sub < 30)
    posw = jnp.where(in_band[None, :] & (a_of[None, :] == jnp.arange(8)[:, None]),
                     omega[f_of][None, :], 0.0).astype(f32)

    x, gidx, tab = _run_topk(xyz128, features, W1, b1.reshape(1, D_MODEL), posw)

    kf, ktab = _gather_sc(x.reshape(B * N, D_MODEL),
                          tab.reshape(B * N, 128),
                          gidx.reshape(ROWS))

    c32 = jnp.arange(32)
    va = c32 < 30
    a32 = jnp.clip(c32 // 10, 0, 2)
    f32_ = c32 % 10
    Wd1s = jnp.where(va[:, None], Wd1[jnp.clip(20 * a32 + f32_, 0, POS_DIM - 1)], 0.0)
    Wd1c = jnp.where(va[:, None], Wd1[jnp.clip(20 * a32 + 10 + f32_, 0, POS_DIM - 1)], 0.0)

    attn, res = _run_mlp(x, tab, kf, ktab, Wd1s, Wd1c,
                         bd1.reshape(1, D_MODEL), Wd2, bd2.reshape(1, D_MODEL),
                         Wg1, bg1.reshape(1, D_MODEL), Wg2, bg2.reshape(1, D_MODEL),
                         W2, b2.reshape(1, D_MODEL))
    return (res, attn)


# speedup vs baseline: 32.3446x; 1.3285x over previous
import functools

import jax
import jax.numpy as jnp
from jax import lax
from jax.experimental import pallas as pl
from jax.experimental.pallas import tpu as pltpu
from jax.experimental.pallas import tpu_sc as plsc

D_POINTS = 64
D_MODEL = 128
K = 24
POS_DIM = 60

B = 4
N = 4096
MA = 512
MC = 256
NSLAB = N // 128
IMAX = 0x7FFFFFFF


def _bdot(a, b, dims=(((1,), (0,)), ((), ()))):
    return lax.dot_general(a.astype(jnp.bfloat16), b.astype(jnp.bfloat16),
                           dims, preferred_element_type=jnp.float32)


def _topk_body(xyzq_ref, xyzk_ref, feat_ref, w1_ref, b1_ref, posw_ref,
               x_ref, idx_ref, tab_ref):
    q = xyzq_ref[0]
    kx = xyzk_ref[0]
    q2 = jnp.sum(q * q, axis=-1)
    k2 = jnp.sum(kx * kx, axis=-1)
    qk = _bdot(q, kx, (((1,), (1,)), ((), ())))
    d = q2[:, None] + k2[None, :] - 2.0 * qk

    bits = lax.bitcast_convert_type(d, jnp.int32)
    ikey = bits ^ (lax.shift_right_arithmetic(bits, 31) & jnp.int32(0x7FFFFFFF))

    m1 = m2 = m3 = m4 = jnp.full((MA, 128), IMAX, jnp.int32)
    for j in range(NSLAB):
        s = (ikey[:, j * 128:(j + 1) * 128] & jnp.int32(-32)) | jnp.int32(j)
        lo = jnp.minimum(m1, s)
        hi = jnp.maximum(m1, s)
        m1 = lo
        lo = jnp.minimum(m2, hi)
        hi = jnp.maximum(m2, hi)
        m2 = lo
        lo = jnp.minimum(m3, hi)
        hi = jnp.maximum(m3, hi)
        m3 = lo
        m4 = jnp.minimum(m4, hi)

    lane = lax.broadcasted_iota(jnp.int32, (MA, 128), 1)
    kcol = lax.broadcasted_iota(jnp.int32, (MA, K), 1)
    out_idx = jnp.zeros((MA, K), jnp.int32)
    for kk in range(K):
        w = jnp.min(m1, axis=1, keepdims=True)
        eq = m1 == w
        l = jnp.min(jnp.where(eq, lane, jnp.int32(128)), axis=1, keepdims=True)
        gidx = (w & jnp.int32(31)) * 128 + l
        out_idx = jnp.where(kcol == kk, gidx, out_idx)
        eql = lane == l
        m1 = jnp.where(eql, m2, m1)
        m2 = jnp.where(eql, m3, m2)
        m3 = jnp.where(eql, m4, m3)
        m4 = jnp.where(eql, IMAX, m4)

    idx_ref[0] = out_idx

    posw = posw_ref[...]
    ang = (q[:, 0:1] * posw[0:1, :]
           + q[:, 1:2] * posw[1:2, :]
           + q[:, 2:3] * posw[2:3, :])
    sn = jnp.sin(ang)
    cs = jnp.cos(ang)
    tab = jnp.where(lane < 32, q, jnp.where(lane < 64, sn,
                    jnp.where(lane < 96, cs, 0.0)))
    tab_ref[0] = tab

    x_ref[0] = _bdot(feat_ref[0], w1_ref[...]) + b1_ref[...]


def _run_topk(xyz128, features, W1, b1, posw):
    nb = xyz128.shape[0]
    grid = (nb, N // MA)
    return pl.pallas_call(
        _topk_body,
        grid=grid,
        in_specs=[
            pl.BlockSpec((1, MA, 128), lambda b, i: (b, i, 0)),
            pl.BlockSpec((1, N, 128), lambda b, i: (b, 0, 0)),
            pl.BlockSpec((1, MA, D_POINTS), lambda b, i: (b, i, 0)),
            pl.BlockSpec((D_POINTS, D_MODEL), lambda b, i: (0, 0)),
            pl.BlockSpec((1, D_MODEL), lambda b, i: (0, 0)),
            pl.BlockSpec((8, 128), lambda b, i: (0, 0)),
        ],
        out_specs=[
            pl.BlockSpec((1, MA, D_MODEL), lambda b, i: (b, i, 0)),
            pl.BlockSpec((1, MA, K), lambda b, i: (b, i, 0)),
            pl.BlockSpec((1, MA, 128), lambda b, i: (b, i, 0)),
        ],
        out_shape=[
            jax.ShapeDtypeStruct((nb, N, D_MODEL), jnp.float32),
            jax.ShapeDtypeStruct((nb, N, K), jnp.int32),
            jax.ShapeDtypeStruct((nb, N, 128), jnp.float32),
        ],
        compiler_params=pltpu.CompilerParams(
            dimension_semantics=("parallel", "parallel")),
    )(xyz128, xyz128, features, W1, b1, posw)


NC_SC = 2
NS_SC = 16
NW = NC_SC * NS_SC
ROWS = B * N * K
RPW = ROWS // NW
CH = 128
NCHUNK = RPW // CH


def _gather_sc(x2, tab2, gidx):
    rows = gidx.shape[0]
    rpw = rows // NW
    nchunk = rpw // CH
    mesh = plsc.VectorSubcoreMesh(core_axis_name="c", subcore_axis_name="s")

    @functools.partial(
        pl.kernel, mesh=mesh,
        out_type=[
            jax.ShapeDtypeStruct((rows, D_MODEL), jnp.float32),
            jax.ShapeDtypeStruct((rows, 128), jnp.float32),
        ],
        scratch_types=[
            pltpu.VMEM((CH,), jnp.int32),
            pltpu.VMEM((CH, D_MODEL), jnp.float32),
            pltpu.VMEM((CH, 128), jnp.float32),
            pltpu.SemaphoreType.DMA,
            pltpu.SemaphoreType.DMA,
        ],
    )
    def gather_kernel(x2_hbm, tab_hbm, gidx_hbm, kf_hbm, ktab_hbm,
                      idxv, kfv, ktv, sem0, sem1):
        wid = lax.axis_index("s") * NC_SC + lax.axis_index("c")
        base0 = wid * rpw

        def body(i, carry):
            base = base0 + i * CH
            pltpu.sync_copy(gidx_hbm.at[pl.ds(base, CH)], idxv)
            cp0 = pltpu.async_copy(x2_hbm.at[idxv], kfv, sem0)
            cp1 = pltpu.async_copy(tab_hbm.at[idxv], ktv, sem1)
            cp0.wait()
            cp1.wait()
            pltpu.sync_copy(kfv, kf_hbm.at[pl.ds(base, CH)])
            pltpu.sync_copy(ktv, ktab_hbm.at[pl.ds(base, CH)])
            return carry

        lax.fori_loop(0, nchunk, body, 0)

    return gather_kernel(x2, tab2, gidx)


def _mlp_body(x_ref, tabq_ref, kf_ref, kt_ref,
              wd1s_ref, wd1c_ref, bd1_ref, wd2_ref, bd2_ref,
              wg1_ref, bg1_ref, wg2_ref, bg2_ref, w2_ref, b2_ref,
              attn_ref, res_ref):
    MK = MC * K
    xq = x_ref[0]
    tq = tabq_ref[0]
    kf = kf_ref[...]
    kt = kt_ref[...]

    tq_rep = jnp.broadcast_to(tq[:, None, :], (MC, K, 128)).reshape(MK, 128)
    qsin = tq_rep[:, 32:64]
    qcos = tq_rep[:, 64:96]
    ksin = kt[:, 32:64]
    kcos = kt[:, 64:96]
    emb_sin = qsin * kcos - qcos * ksin
    emb_cos = qcos * kcos + qsin * ksin

    pos_pre = _bdot(emb_sin, wd1s_ref[...]) + _bdot(emb_cos, wd1c_ref[...]) + bd1_ref[...]
    pos_enc = _bdot(jnp.maximum(pos_pre, 0.0), wd2_ref[...]) + bd2_ref[...]

    xq_rep = jnp.broadcast_to(xq[:, None, :], (MC, K, D_MODEL)).reshape(MK, D_MODEL)
    pre = xq_rep - kf + pos_enc
    a1 = _bdot(jnp.maximum(_bdot(pre, wg1_ref[...]) + bg1_ref[...], 0.0),
               wg2_ref[...]) + bg2_ref[...]
    a1 = a1 / jnp.sqrt(jnp.float32(D_MODEL))

    a3 = a1.reshape(MC, K, D_MODEL)
    mx = jnp.max(a3, axis=1, keepdims=True)
    e = jnp.exp(a3 - mx)
    s = jnp.sum(e, axis=1, keepdims=True)
    attn = e / s
    attn_ref[0] = attn

    prod = attn.reshape(MK, D_MODEL) * (kf + pos_enc)
    red = jnp.sum(prod.reshape(MC, K, D_MODEL), axis=1)
    res_ref[0] = _bdot(red, w2_ref[...]) + b2_ref[...] + xq


def _run_mlp(x, tab, kf, ktab, Wd1s, Wd1c, bd1, Wd2, bd2, Wg1, bg1, Wg2, bg2, W2, b2):
    nb = x.shape[0]
    grid = (nb, N // MC)
    nblk = N // MC
    return pl.pallas_call(
        _mlp_body,
        grid=grid,
        in_specs=[
            pl.BlockSpec((1, MC, D_MODEL), lambda b, i: (b, i, 0)),
            pl.BlockSpec((1, MC, 128), lambda b, i: (b, i, 0)),
            pl.BlockSpec((MC * K, D_MODEL), lambda b, i: (b * nblk + i, 0)),
            pl.BlockSpec((MC * K, 128), lambda b, i: (b * nblk + i, 0)),
            pl.BlockSpec((32, D_MODEL), lambda b, i: (0, 0)),
            pl.BlockSpec((32, D_MODEL), lambda b, i: (0, 0)),
            pl.BlockSpec((1, D_MODEL), lambda b, i: (0, 0)),
            pl.BlockSpec((D_MODEL, D_MODEL), lambda b, i: (0, 0)),
            pl.BlockSpec((1, D_MODEL), lambda b, i: (0, 0)),
            pl.BlockSpec((D_MODEL, D_MODEL), lambda b, i: (0, 0)),
            pl.BlockSpec((1, D_MODEL), lambda b, i: (0, 0)),
            pl.BlockSpec((D_MODEL, D_MODEL), lambda b, i: (0, 0)),
            pl.BlockSpec((1, D_MODEL), lambda b, i: (0, 0)),
            pl.BlockSpec((D_MODEL, D_MODEL), lambda b, i: (0, 0)),
            pl.BlockSpec((1, D_MODEL), lambda b, i: (0, 0)),
        ],
        out_specs=[
            pl.BlockSpec((1, MC, K, D_MODEL), lambda b, i: (b, i, 0, 0)),
            pl.BlockSpec((1, MC, D_MODEL), lambda b, i: (b, i, 0)),
        ],
        out_shape=[
            jax.ShapeDtypeStruct((nb, N, K, D_MODEL), jnp.float32),
            jax.ShapeDtypeStruct((nb, N, D_MODEL), jnp.float32),
        ],
        compiler_params=pltpu.CompilerParams(
            dimension_semantics=("parallel", "parallel")),
    )(x, tab, kf, ktab, Wd1s, Wd1c, bd1, Wd2, bd2, Wg1, bg1, Wg2, bg2, W2, b2)


def kernel(features, xyz, W1, b1, W2, b2, Wd1, bd1, Wd2, bd2, Wg1, bg1, Wg2, bg2):
    f32 = jnp.float32
    xyz128 = jnp.concatenate(
        [xyz, jnp.zeros((B, N, 125), f32)], axis=-1)

    d2 = POS_DIM // 3
    omega = jnp.arange(d2 // 2, dtype=f32) / (d2 / 2.0)
    omega = 1.0 / (10000.0 ** omega)
    col = jnp.arange(128)
    sub = col % 32
    a_of = jnp.clip(sub // 10, 0, 2)
    f_of = sub % 10
    in_band = ((col >= 32) & (col < 128 - 32)) & (sub < 30)
    posw = jnp.where(in_band[None, :] & (a_of[None, :] == jnp.arange(8)[:, None]),
                     omega[f_of][None, :], 0.0).astype(f32)

    c32 = jnp.arange(32)
    va = c32 < 30
    a32 = jnp.clip(c32 // 10, 0, 2)
    f32_ = c32 % 10
    Wd1s = jnp.where(va[:, None], Wd1[jnp.clip(20 * a32 + f32_, 0, POS_DIM - 1)], 0.0)
    Wd1c = jnp.where(va[:, None], Wd1[jnp.clip(20 * a32 + 10 + f32_, 0, POS_DIM - 1)], 0.0)

    attn_parts, res_parts = [], []
    for bb in range(B):
        xb, gidxb, tabb = _run_topk(xyz128[bb:bb + 1], features[bb:bb + 1],
                                    W1, b1.reshape(1, D_MODEL), posw)
        kfb, ktabb = _gather_sc(xb.reshape(N, D_MODEL), tabb.reshape(N, 128),
                                gidxb.reshape(N * K))
        attnb, resb = _run_mlp(xb, tabb, kfb, ktabb, Wd1s, Wd1c,
                               bd1.reshape(1, D_MODEL), Wd2, bd2.reshape(1, D_MODEL),
                               Wg1, bg1.reshape(1, D_MODEL), Wg2, bg2.reshape(1, D_MODEL),
                               W2, b2.reshape(1, D_MODEL))
        attn_parts.append(attnb)
        res_parts.append(resb)
    attn = jnp.concatenate(attn_parts, axis=0)
    res = jnp.concatenate(res_parts, axis=0)
    return (res, attn)
